# baseline (speedup 1.0000x reference)
import jax
import jax.numpy as jnp
from jax.experimental import pallas as pl
from jax.experimental.pallas import tpu as pltpu

B, N, D = 8, 2048, 32


def _hidden_kernel(text_ref, w_ref, out_ref):
    h = jnp.dot(
        text_ref[...].reshape(B * N, D),
        w_ref[...],
        preferred_element_type=jnp.float32,
    )
    out_ref[:, :D] = h
    out_ref[:, D:] = jnp.ones((B * N, D), jnp.float32)


def _gcn_stream_kernel(adj_ref, hid_ref, b_ref, out_ref):
    a = adj_ref[0]
    acc = jnp.dot(a, hid_ref[0], preferred_element_type=jnp.float32)
    denom = acc[:, D : D + 1] + 1.0
    out_ref[0] = acc[:, :D] / denom + b_ref[...]


def kernel(text, adj, W, b):
    hidden_aug = pl.pallas_call(
        _hidden_kernel,
        out_shape=jax.ShapeDtypeStruct((B * N, 2 * D), jnp.float32),
    )(text, W)
    hidden_aug = hidden_aug.reshape(B, N, 2 * D)

    b2d = b.reshape(1, D)
    return pl.pallas_call(
        _gcn_stream_kernel,
        grid=(B,),
        in_specs=[
            pl.BlockSpec((1, N, N), lambda bi: (bi, 0, 0)),
            pl.BlockSpec((1, N, 2 * D), lambda bi: (bi, 0, 0)),
            pl.BlockSpec((1, D), lambda bi: (0, 0)),
        ],
        out_specs=pl.BlockSpec((1, N, D), lambda bi: (bi, 0, 0)),
        out_shape=jax.ShapeDtypeStruct((B, N, D), jnp.float32),
        compiler_params=pltpu.CompilerParams(
            dimension_semantics=("arbitrary",),
        ),
    )(adj, hidden_aug, b2d)


# speedup vs baseline: 1.1359x; 1.1359x over previous
import jax
import jax.numpy as jnp
from jax.experimental import pallas as pl
from jax.experimental.pallas import tpu as pltpu

B, N, D = 8, 2048, 32


def _gcn_fused_kernel(text_ref, adj_ref, w_ref, b_ref, out_ref, hidden_ref):
    bi = pl.program_id(0)

    @pl.when(bi == 0)
    def _():
        hidden_ref[0, :, :D] = jnp.dot(
            text_ref[0], w_ref[...], preferred_element_type=jnp.float32
        )
        hidden_ref[0, :, D:] = jnp.ones((N, D), jnp.float32)

    a = adj_ref[0]
    acc = jnp.dot(a, hidden_ref[bi % 2], preferred_element_type=jnp.float32)
    out_ref[0] = acc[:, :D] / (acc[:, D:] + 1.0) + b_ref[...]

    @pl.when(bi + 1 < B)
    def _():
        nxt = (bi + 1) % 2
        hidden_ref[nxt, :, :D] = jnp.dot(
            text_ref[bi + 1], w_ref[...], preferred_element_type=jnp.float32
        )
        hidden_ref[nxt, :, D:] = jnp.ones((N, D), jnp.float32)


def kernel(text, adj, W, b):
    b2d = b.reshape(1, D)
    return pl.pallas_call(
        _gcn_fused_kernel,
        grid=(B,),
        in_specs=[
            pl.BlockSpec((B, N, D), lambda bi: (0, 0, 0)),
            pl.BlockSpec((1, N, N), lambda bi: (bi, 0, 0)),
            pl.BlockSpec((D, D), lambda bi: (0, 0)),
            pl.BlockSpec((1, D), lambda bi: (0, 0)),
        ],
        out_specs=pl.BlockSpec((1, N, D), lambda bi: (bi, 0, 0)),
        out_shape=jax.ShapeDtypeStruct((B, N, D), jnp.float32),
        scratch_shapes=[pltpu.VMEM((2, N, 2 * D), jnp.float32)],
        compiler_params=pltpu.CompilerParams(
            dimension_semantics=("arbitrary",),
        ),
    )(text, adj, W, b2d)


# speedup vs baseline: 1.1435x; 1.0066x over previous
import jax
import jax.numpy as jnp
from jax.experimental import pallas as pl
from jax.experimental.pallas import tpu as pltpu

B, N, D = 8, 2048, 32


def _gcn_fused_kernel(text_ref, adj_ref, w_ref, b_ref, out_ref, hidden_ref):
    bi = pl.program_id(0)

    @pl.when(bi == 0)
    def _():
        hidden_ref[:, :D] = jnp.dot(
            text_ref[...], w_ref[...], preferred_element_type=jnp.float32
        )
        hidden_ref[:, D:] = jnp.ones((B * N, D), jnp.float32)

    a = adj_ref[0]
    h = hidden_ref[pl.ds(bi * N, N), :]
    acc = jnp.dot(a, h, preferred_element_type=jnp.float32)
    out_ref[0] = acc[:, :D] / (acc[:, D:] + 1.0) + b_ref[...]


def kernel(text, adj, W, b):
    b2d = b.reshape(1, D)
    text2d = text.reshape(B * N, D)
    return pl.pallas_call(
        _gcn_fused_kernel,
        grid=(B,),
        in_specs=[
            pl.BlockSpec((B * N, D), lambda bi: (0, 0)),
            pl.BlockSpec((1, N, N), lambda bi: (bi, 0, 0)),
            pl.BlockSpec((D, D), lambda bi: (0, 0)),
            pl.BlockSpec((1, D), lambda bi: (0, 0)),
        ],
        out_specs=pl.BlockSpec((1, N, D), lambda bi: (bi, 0, 0)),
        out_shape=jax.ShapeDtypeStruct((B, N, D), jnp.float32),
        scratch_shapes=[pltpu.VMEM((B * N, 2 * D), jnp.float32)],
        compiler_params=pltpu.CompilerParams(
            dimension_semantics=("arbitrary",),
        ),
    )(text2d, adj, W, b2d)
